# baseline (speedup 1.0000x reference)
import jax
import jax.numpy as jnp
from jax.experimental import pallas as pl
from jax.experimental.pallas import tpu as pltpu

DIM = 128
NUM_HEADS = 8
N_WIN = 7
TOPK = 4
HEAD_DIM = DIM // NUM_HEADS
SCALE = DIM ** -0.5
NREG = N_WIN ** 3
RSS = 64
SEQ = NREG * RSS
KV = TOPK * RSS
SLAB = 4 * 28 * 28
HPAD = 36

_INTERPRET = False


def _qkv_kernel(x_ref, w_ref, b_ref, q_ref, k_ref, v_ref, vr_ref, pool_ref):
    a = pl.program_id(0)
    y = jnp.dot(x_ref[:], w_ref[:], preferred_element_type=jnp.float32) + b_ref[:]
    q_ref[:] = y[:, :DIM]
    k_ref[:] = y[:, DIM:2 * DIM]
    yv = y[:, 2 * DIM:]
    v_ref[:] = yv
    p = y[:, :2 * DIM].reshape(N_WIN, RSS, 2 * DIM)
    pool_ref[0] = jnp.sum(p, axis=1) * (1.0 / RSS)
    vr = yv.reshape(N_WIN, 4, 4, 4, DIM).transpose(1, 2, 0, 3, 4)
    vr = vr.reshape(4, 4, 28, DIM)
    live = ((a >= 1) & (a <= 7)).astype(jnp.float32)
    vr_ref[:] = vr * live


def _qkv_call(x_seq, w_t, b2d):
    m_blk = N_WIN * RSS
    blk = lambda a, b: (jnp.clip(a - 1, 0, 6) * N_WIN + b, 0)
    return pl.pallas_call(
        _qkv_kernel,
        grid=(9, 7),
        in_specs=[
            pl.BlockSpec((m_blk, DIM), blk),
            pl.BlockSpec((DIM, 3 * DIM), lambda a, b: (0, 0)),
            pl.BlockSpec((1, 3 * DIM), lambda a, b: (0, 0)),
        ],
        out_specs=[
            pl.BlockSpec((m_blk, DIM), blk),
            pl.BlockSpec((m_blk, DIM), blk),
            pl.BlockSpec((m_blk, DIM), blk),
            pl.BlockSpec((4, 4, 28, DIM), lambda a, b: (a, b, 0, 0)),
            pl.BlockSpec((1, N_WIN, 2 * DIM),
                         lambda a, b: (jnp.clip(a - 1, 0, 6) * N_WIN + b, 0, 0)),
        ],
        out_shape=[
            jax.ShapeDtypeStruct((SEQ, DIM), jnp.float32),
            jax.ShapeDtypeStruct((SEQ, DIM), jnp.float32),
            jax.ShapeDtypeStruct((SEQ, DIM), jnp.float32),
            jax.ShapeDtypeStruct((HPAD, 28, 28, DIM), jnp.float32),
            jax.ShapeDtypeStruct((49, N_WIN, 2 * DIM), jnp.float32),
        ],
        interpret=_INTERPRET,
    )(x_seq, w_t, b2d)


def _route_kernel(pool_ref, idx_ref):
    qp = pool_ref[:, :DIM]
    kp = pool_ref[:, DIM:]
    a = jax.lax.dot_general(qp, kp, (((1,), (1,)), ((), ())),
                            preferred_element_type=jnp.float32)
    col = jax.lax.broadcasted_iota(jnp.int32, a.shape, 1)
    for j in range(TOPK):
        m = jnp.max(a, axis=1, keepdims=True)
        cand = jnp.where(a >= m, col, NREG + 1)
        sel = jnp.min(cand, axis=1, keepdims=True)
        idx_ref[:, j:j + 1] = sel
        a = jnp.where(col == sel, -jnp.inf, a)


def _route_call(pools):
    return pl.pallas_call(
        _route_kernel,
        out_shape=jax.ShapeDtypeStruct((NREG, TOPK), jnp.int32),
        interpret=_INTERPRET,
    )(pools)


def _attn_kernel(idx_ref, q_ref, k_ref, v_ref, o_ref):
    a, b, c = pl.program_id(0), pl.program_id(1), pl.program_id(2)
    r = (a * N_WIN + b) * N_WIN + c
    q = q_ref[:] * SCALE
    ks = [k_ref[pl.ds(idx_ref[r, j] * RSS, RSS), :] for j in range(TOPK)]
    vs = [v_ref[pl.ds(idx_ref[r, j] * RSS, RSS), :] for j in range(TOPK)]
    kg = jnp.concatenate(ks, axis=0)
    vg = jnp.concatenate(vs, axis=0)
    big = NUM_HEADS * RSS
    qt = jnp.broadcast_to(q[None], (NUM_HEADS, RSS, DIM)).reshape(big, DIM)
    rowh = jax.lax.broadcasted_iota(jnp.int32, (big, DIM), 0) // RSS
    colh = jax.lax.broadcasted_iota(jnp.int32, (big, DIM), 1) // HEAD_DIM
    qbd = jnp.where(rowh == colh, qt, 0.0).astype(jnp.bfloat16)
    s = jax.lax.dot_general(qbd, kg.astype(jnp.bfloat16),
                            (((1,), (1,)), ((), ())),
                            preferred_element_type=jnp.float32)
    s = s - jnp.max(s, axis=1, keepdims=True)
    e = jnp.exp(s)
    denom = jnp.sum(e, axis=1, keepdims=True)
    o3 = jnp.dot(e.astype(jnp.bfloat16), vg.astype(jnp.bfloat16),
                 preferred_element_type=jnp.float32)
    o3 = o3 / denom
    hsel = jax.lax.broadcasted_iota(jnp.int32, (RSS, DIM), 1) // HEAD_DIM
    acc = jnp.zeros((RSS, DIM), jnp.float32)
    for m in range(NUM_HEADS):
        acc = acc + jnp.where(hsel == m, o3[m * RSS:(m + 1) * RSS, :], 0.0)
    o_ref[:, :, pl.ds(c * 4, 4), :] = acc.reshape(4, 4, 4, DIM)


def _attn_call(q, k, v, idx):
    return pl.pallas_call(
        _attn_kernel,
        grid=(N_WIN, N_WIN, N_WIN),
        in_specs=[
            pl.BlockSpec(memory_space=pltpu.SMEM),
            pl.BlockSpec((RSS, DIM),
                         lambda a, b, c: ((a * N_WIN + b) * N_WIN + c, 0)),
            pl.BlockSpec((SEQ, DIM), lambda a, b, c: (0, 0)),
            pl.BlockSpec((SEQ, DIM), lambda a, b, c: (0, 0)),
        ],
        out_specs=pl.BlockSpec((4, 4, 28, DIM), lambda a, b, c: (a, b, 0, 0)),
        out_shape=jax.ShapeDtypeStruct((28, 28, 28, DIM), jnp.float32),
        interpret=_INTERPRET,
    )(idx, q, k, v)


def _fin_kernel(v0_ref, v1_ref, v2_ref, v3_ref, at_ref, w_ref, bl_ref,
                wo_ref, bo_ref, o_ref):
    vp = jnp.concatenate([v0_ref[:], v1_ref[:], v2_ref[:], v3_ref[:]], axis=0)
    n = jax.lax.broadcasted_iota(jnp.int32, (SLAB, 1), 0)
    w_pos = (n // 28) % 28
    d_pos = n % 28
    wmask = {-1: (w_pos > 0), 0: (w_pos >= 0), 1: (w_pos < 27)}
    dmask = {-1: (d_pos > 0), 0: (d_pos >= 0), 1: (d_pos < 27)}
    acc = at_ref[:] + bl_ref[:]
    for t in range(27):
        dh, dw, du = t // 9 - 1, (t // 3) % 3 - 1, t % 3 - 1
        delta = 784 * dh + 28 * dw + du
        sl = vp[1568 + delta:1568 + delta + SLAB, :]
        if dw == 0 and du == 0:
            acc = acc + sl * w_ref[t:t + 1, :]
        else:
            m = (wmask[dw] & dmask[du]).astype(jnp.float32)
            acc = acc + sl * w_ref[t:t + 1, :] * m
    o_ref[:] = jnp.dot(acc, wo_ref[:],
                       preferred_element_type=jnp.float32) + bo_ref[:]


def _fin_call(vp_flat, attn_flat, w27, bl2d, wo_t, bo2d):
    return pl.pallas_call(
        _fin_kernel,
        grid=(N_WIN,),
        in_specs=[
            pl.BlockSpec((1568, DIM), lambda a: (2 * a + 1, 0)),
            pl.BlockSpec((1568, DIM), lambda a: (2 * a + 2, 0)),
            pl.BlockSpec((1568, DIM), lambda a: (2 * a + 3, 0)),
            pl.BlockSpec((1568, DIM), lambda a: (2 * a + 4, 0)),
            pl.BlockSpec((SLAB, DIM), lambda a: (a, 0)),
            pl.BlockSpec((27, DIM), lambda a: (0, 0)),
            pl.BlockSpec((1, DIM), lambda a: (0, 0)),
            pl.BlockSpec((DIM, DIM), lambda a: (0, 0)),
            pl.BlockSpec((1, DIM), lambda a: (0, 0)),
        ],
        out_specs=pl.BlockSpec((SLAB, DIM), lambda a: (a, 0)),
        out_shape=jax.ShapeDtypeStruct((SEQ, DIM), jnp.float32),
        interpret=_INTERPRET,
    )(vp_flat, vp_flat, vp_flat, vp_flat, attn_flat, w27, bl2d, wo_t, bo2d)


def kernel(x, W_qkv, b_qkv, W_lepe, b_lepe, W_out, b_out):
    C, H, W_, D = DIM, 28, 28, 28
    rs = H // N_WIN
    xt = x[0].reshape(C, N_WIN, rs, N_WIN, rs, N_WIN, rs)
    xt = jnp.transpose(xt, (1, 3, 5, 2, 4, 6, 0)).reshape(SEQ, C)

    q, k, v, v_raster, pools = _qkv_call(xt, W_qkv.T, b_qkv[None, :])
    idx = _route_call(pools.reshape(NREG, 2 * C))
    attn_raster = _attn_call(q, k, v, idx)

    vp_flat = v_raster.reshape(HPAD * 28 * 28, C)
    out_flat = _fin_call(vp_flat, attn_raster.reshape(SEQ, C),
                         W_lepe.reshape(C, 27).T, b_lepe[None, :],
                         W_out.T, b_out[None, :])
    out = jnp.transpose(out_flat.reshape(H, W_, D, C), (3, 0, 1, 2))
    return out[None]


# speedup vs baseline: 1.0946x; 1.0946x over previous
import jax
import jax.numpy as jnp
from jax.experimental import pallas as pl
from jax.experimental.pallas import tpu as pltpu

DIM = 128
NUM_HEADS = 8
N_WIN = 7
TOPK = 4
HEAD_DIM = DIM // NUM_HEADS
SCALE = DIM ** -0.5
NREG = N_WIN ** 3
RSS = 64
SEQ = NREG * RSS
KV = TOPK * RSS
SLAB = 4 * 28 * 28
HPAD = 36

_INTERPRET = False


_QKV_RBLK = 49


def _qkv_kernel(x_ref, w_ref, b_ref, q_ref, k_ref, v_ref, pool_ref):
    y = jnp.dot(x_ref[:], w_ref[:], preferred_element_type=jnp.float32) + b_ref[:]
    q_ref[:] = y[:, :DIM]
    k_ref[:] = y[:, DIM:2 * DIM]
    v_ref[:] = y[:, 2 * DIM:]
    p = y[:, :2 * DIM].reshape(_QKV_RBLK, RSS, 2 * DIM)
    pool_ref[0] = jnp.sum(p, axis=1) * (1.0 / RSS)


def _qkv_call(x_seq, w_t, b2d):
    m_blk = _QKV_RBLK * RSS
    seq_spec = pl.BlockSpec((m_blk, DIM), lambda i: (i, 0))
    return pl.pallas_call(
        _qkv_kernel,
        grid=(NREG // _QKV_RBLK,),
        in_specs=[
            seq_spec,
            pl.BlockSpec((DIM, 3 * DIM), lambda i: (0, 0)),
            pl.BlockSpec((1, 3 * DIM), lambda i: (0, 0)),
        ],
        out_specs=[
            seq_spec, seq_spec, seq_spec,
            pl.BlockSpec((1, _QKV_RBLK, 2 * DIM), lambda i: (i, 0, 0)),
        ],
        out_shape=[
            jax.ShapeDtypeStruct((SEQ, DIM), jnp.float32),
            jax.ShapeDtypeStruct((SEQ, DIM), jnp.float32),
            jax.ShapeDtypeStruct((SEQ, DIM), jnp.float32),
            jax.ShapeDtypeStruct((NREG // _QKV_RBLK, _QKV_RBLK, 2 * DIM), jnp.float32),
        ],
        interpret=_INTERPRET,
    )(x_seq, w_t, b2d)


def _route_kernel(pool_ref, idx_ref):
    qp = pool_ref[:, :DIM]
    kp = pool_ref[:, DIM:]
    a = jax.lax.dot_general(qp, kp, (((1,), (1,)), ((), ())),
                            preferred_element_type=jnp.float32)
    col = jax.lax.broadcasted_iota(jnp.int32, a.shape, 1)
    for j in range(TOPK):
        m = jnp.max(a, axis=1, keepdims=True)
        cand = jnp.where(a >= m, col, NREG + 1)
        sel = jnp.min(cand, axis=1, keepdims=True)
        idx_ref[:, j:j + 1] = sel
        a = jnp.where(col == sel, -jnp.inf, a)


def _route_call(pools):
    return pl.pallas_call(
        _route_kernel,
        out_shape=jax.ShapeDtypeStruct((NREG, TOPK), jnp.int32),
        interpret=_INTERPRET,
    )(pools)


def _attn_kernel(idx_ref, q_ref, k_ref, v_ref, o_ref):
    r = pl.program_id(0)
    q = q_ref[:] * SCALE
    ks = [k_ref[pl.ds(idx_ref[r, j] * RSS, RSS), :] for j in range(TOPK)]
    vs = [v_ref[pl.ds(idx_ref[r, j] * RSS, RSS), :] for j in range(TOPK)]
    kg = jnp.concatenate(ks, axis=0)
    vg = jnp.concatenate(vs, axis=0)
    big = NUM_HEADS * RSS
    qt = jnp.broadcast_to(q[None], (NUM_HEADS, RSS, DIM)).reshape(big, DIM)
    rowh = jax.lax.broadcasted_iota(jnp.int32, (big, DIM), 0) // RSS
    colh = jax.lax.broadcasted_iota(jnp.int32, (big, DIM), 1) // HEAD_DIM
    qbd = jnp.where(rowh == colh, qt, 0.0).astype(jnp.bfloat16)
    s = jax.lax.dot_general(qbd, kg.astype(jnp.bfloat16),
                            (((1,), (1,)), ((), ())),
                            preferred_element_type=jnp.float32)
    s = s - jnp.max(s, axis=1, keepdims=True)
    e = jnp.exp(s)
    denom = jnp.sum(e, axis=1, keepdims=True)
    o3 = jnp.dot(e.astype(jnp.bfloat16), vg.astype(jnp.bfloat16),
                 preferred_element_type=jnp.float32)
    o3 = o3 / denom
    hsel = jax.lax.broadcasted_iota(jnp.int32, (RSS, DIM), 1) // HEAD_DIM
    acc = jnp.zeros((RSS, DIM), jnp.float32)
    for m in range(NUM_HEADS):
        acc = acc + jnp.where(hsel == m, o3[m * RSS:(m + 1) * RSS, :], 0.0)
    o_ref[:] = acc


def _attn_call(q, k, v, idx):
    return pl.pallas_call(
        _attn_kernel,
        grid=(NREG,),
        in_specs=[
            pl.BlockSpec(memory_space=pltpu.SMEM),
            pl.BlockSpec((RSS, DIM), lambda r: (r, 0)),
            pl.BlockSpec((SEQ, DIM), lambda r: (0, 0)),
            pl.BlockSpec((SEQ, DIM), lambda r: (0, 0)),
        ],
        out_specs=pl.BlockSpec((RSS, DIM), lambda r: (r, 0)),
        out_shape=jax.ShapeDtypeStruct((SEQ, DIM), jnp.float32),
        interpret=_INTERPRET,
    )(idx, q, k, v)


def _fin_kernel(v0_ref, v1_ref, v2_ref, v3_ref, at_ref, w_ref, bl_ref,
                wo_ref, bo_ref, o_ref):
    vp = jnp.concatenate([v0_ref[:], v1_ref[:], v2_ref[:], v3_ref[:]], axis=0)
    n = jax.lax.broadcasted_iota(jnp.int32, (SLAB, 1), 0)
    w_pos = (n // 28) % 28
    d_pos = n % 28
    wmask = {-1: (w_pos > 0), 0: (w_pos >= 0), 1: (w_pos < 27)}
    dmask = {-1: (d_pos > 0), 0: (d_pos >= 0), 1: (d_pos < 27)}
    acc = at_ref[:] + bl_ref[:]
    for t in range(27):
        dh, dw, du = t // 9 - 1, (t // 3) % 3 - 1, t % 3 - 1
        delta = 784 * dh + 28 * dw + du
        sl = vp[1568 + delta:1568 + delta + SLAB, :]
        if dw == 0 and du == 0:
            acc = acc + sl * w_ref[t:t + 1, :]
        else:
            m = (wmask[dw] & dmask[du]).astype(jnp.float32)
            acc = acc + sl * w_ref[t:t + 1, :] * m
    o_ref[:] = jnp.dot(acc, wo_ref[:],
                       preferred_element_type=jnp.float32) + bo_ref[:]


def _fin_call(vp_flat, attn_flat, w27, bl2d, wo_t, bo2d):
    return pl.pallas_call(
        _fin_kernel,
        grid=(N_WIN,),
        in_specs=[
            pl.BlockSpec((1568, DIM), lambda a: (2 * a, 0)),
            pl.BlockSpec((1568, DIM), lambda a: (2 * a + 1, 0)),
            pl.BlockSpec((1568, DIM), lambda a: (2 * a + 2, 0)),
            pl.BlockSpec((1568, DIM), lambda a: (2 * a + 3, 0)),
            pl.BlockSpec((SLAB, DIM), lambda a: (a, 0)),
            pl.BlockSpec((27, DIM), lambda a: (0, 0)),
            pl.BlockSpec((1, DIM), lambda a: (0, 0)),
            pl.BlockSpec((DIM, DIM), lambda a: (0, 0)),
            pl.BlockSpec((1, DIM), lambda a: (0, 0)),
        ],
        out_specs=pl.BlockSpec((SLAB, DIM), lambda a: (a, 0)),
        out_shape=jax.ShapeDtypeStruct((SEQ, DIM), jnp.float32),
        interpret=_INTERPRET,
    )(vp_flat, vp_flat, vp_flat, vp_flat, attn_flat, w27, bl2d, wo_t, bo2d)


def kernel(x, W_qkv, b_qkv, W_lepe, b_lepe, W_out, b_out):
    C, H, W_, D = DIM, 28, 28, 28
    rs = H // N_WIN
    xt = x[0].reshape(C, N_WIN, rs, N_WIN, rs, N_WIN, rs)
    xt = jnp.transpose(xt, (1, 3, 5, 2, 4, 6, 0)).reshape(SEQ, C)

    q, k, v, pools = _qkv_call(xt, W_qkv.T, b_qkv[None, :])
    idx = _route_call(pools.reshape(NREG, 2 * C))
    attn_seq = _attn_call(q, k, v, idx)

    def seq2grid_flat(t):
        t = t.reshape(N_WIN, N_WIN, N_WIN, rs, rs, rs, C)
        t = jnp.transpose(t, (0, 3, 1, 4, 2, 5, 6))
        return t.reshape(SEQ, C)

    vp_flat = jnp.pad(seq2grid_flat(v), ((1568, 1568), (0, 0)))
    out_flat = _fin_call(vp_flat, seq2grid_flat(attn_seq),
                         W_lepe.reshape(C, 27).T, b_lepe[None, :],
                         W_out.T, b_out[None, :])
    out = jnp.transpose(out_flat.reshape(H, W_, D, C), (3, 0, 1, 2))
    return out[None]
